# baseline (speedup 1.0000x reference)
import functools

import jax
import jax.numpy as jnp
from jax import lax
from jax.experimental import pallas as pl
from jax.experimental.pallas import tpu as pltpu
from jax.experimental.pallas import tpu_sc as plsc

N_NODES = 10000
N_PAD = 10240
N_EDGES = 320000
D_IN = 128
D_PAD = 144

NC = 2
NS = 16
NW = NC * NS
E_PER_W = N_EDGES // NW
CHUNK = 80
N_CHUNKS = E_PER_W // CHUNK
N_PAIRS = (N_CHUNKS - 1) // 2
ROWS_PER_S = N_PAD // NS


def _sc_aggregate(xpad, index, index1, zeros):
    mesh = plsc.VectorSubcoreMesh(core_axis_name="c", subcore_axis_name="s")

    @functools.partial(
        pl.kernel,
        mesh=mesh,
        out_type=jax.ShapeDtypeStruct((NC * N_PAD, D_PAD), jnp.float32),
        scratch_types=[
            pltpu.VMEM_SHARED((N_PAD, D_PAD), jnp.float32),
            pltpu.VMEM((2, CHUNK), jnp.int32),
            pltpu.VMEM((N_CHUNKS, CHUNK), jnp.int32),
            pltpu.VMEM((2, CHUNK, D_PAD), jnp.float32),
            pltpu.SemaphoreType.DMA,
            pltpu.SemaphoreType.DMA,
            pltpu.SemaphoreType.DMA,
            pltpu.SemaphoreType.DMA,
        ],
        compiler_params=pltpu.CompilerParams(use_tc_tiling_on_sc=False),
    )
    def k(xpad_h, idx_h, idx1_h, zero_h, out_h,
          acc, idxg, idxs, rows, gs0, gs1, ls0, ls1):
        c = lax.axis_index("c")
        s = lax.axis_index("s")
        wid = c * NS + s

        pltpu.sync_copy(zero_h, acc.at[pl.ds(s * ROWS_PER_S, ROWS_PER_S)])

        ebase = wid * E_PER_W
        pltpu.sync_copy(idx1_h.at[pl.ds(wid * N_CHUNKS, N_CHUNKS)], idxs)
        plsc.subcore_barrier()

        lsem = [ls0, ls1]
        gsem = [gs0, gs1]

        def load_g_start(ch, b):
            pltpu.async_copy(idx_h.at[pl.ds(ebase + ch * CHUNK, CHUNK)],
                             idxg.at[b], lsem[b])

        def load_g_wait(ch, b):
            pltpu.make_async_copy(idx_h.at[pl.ds(ebase + ch * CHUNK, CHUNK)],
                                  idxg.at[b], lsem[b]).wait()

        def gather_start(b):
            pltpu.async_copy(xpad_h.at[idxg.at[b]], rows.at[b], gsem[b])

        def gather_wait(b):
            pltpu.make_async_copy(xpad_h.at[idxg.at[b]], rows.at[b], gsem[b]).wait()

        def scatter(ch, b):
            pltpu.sync_copy(rows.at[b], acc.at[idxs.at[ch]], add=True)

        load_g_start(0, 0)
        load_g_wait(0, 0)
        gather_start(0)
        load_g_start(1, 1)

        def pair_body(i, carry):
            c0 = 2 * i
            c1 = c0 + 1
            gather_wait(0)
            load_g_wait(c1, 1)
            gather_start(1)

            @pl.when(c0 + 2 < N_CHUNKS)
            def _():
                load_g_start(c0 + 2, 0)

            scatter(c0, 0)
            gather_wait(1)

            @pl.when(c0 + 2 < N_CHUNKS)
            def _():
                load_g_wait(c0 + 2, 0)
                gather_start(0)

            @pl.when(c1 + 2 < N_CHUNKS)
            def _():
                load_g_start(c1 + 2, 1)

            scatter(c1, 1)
            return carry

        lax.fori_loop(0, N_PAIRS, pair_body, 0)

        gather_wait(0)
        scatter(N_CHUNKS - 1, 0)
        plsc.subcore_barrier()

        obase = c * N_PAD + s * ROWS_PER_S
        pltpu.sync_copy(acc.at[pl.ds(s * ROWS_PER_S, ROWS_PER_S)],
                        out_h.at[pl.ds(obase, ROWS_PER_S)])

    return k(xpad, index, index1, zeros)


_TC_R = 1000


def _tc_body(x_ref, p0_ref, p1_ref, w_ref, w1_ref, b1_ref, w2_ref, b2_ref, o_ref):
    x = x_ref[...]
    p = p0_ref[0] + p1_ref[0]
    aggr_x = p[:, :D_IN]
    deg = p[:, D_IN:D_IN + 1]
    lin1 = jnp.dot(x, w1_ref[...], preferred_element_type=jnp.float32) + b1_ref[...]
    lin2 = jnp.dot(x, w2_ref[...], preferred_element_type=jnp.float32) + b2_ref[...]
    aggr = jnp.dot(aggr_x, w_ref[...], preferred_element_type=jnp.float32)
    o_ref[...] = deg * lin1 + aggr + lin2


def _tc_finish(x, partial, weight, lin1_w, lin1_b, lin2_w, lin2_b):
    grid = N_NODES // _TC_R
    return pl.pallas_call(
        _tc_body,
        grid=(grid,),
        in_specs=[
            pl.BlockSpec((_TC_R, D_IN), lambda i: (i, 0)),
            pl.BlockSpec((1, _TC_R, D_PAD), lambda i: (0, i, 0)),
            pl.BlockSpec((1, _TC_R, D_PAD), lambda i: (1, i, 0)),
            pl.BlockSpec((D_IN, D_IN), lambda i: (0, 0)),
            pl.BlockSpec((D_IN, D_IN), lambda i: (0, 0)),
            pl.BlockSpec((1, D_IN), lambda i: (0, 0)),
            pl.BlockSpec((D_IN, D_IN), lambda i: (0, 0)),
            pl.BlockSpec((1, D_IN), lambda i: (0, 0)),
        ],
        out_specs=pl.BlockSpec((_TC_R, D_IN), lambda i: (i, 0)),
        out_shape=jax.ShapeDtypeStruct((N_NODES, D_IN), jnp.float32),
    )(x, partial, partial, weight, lin1_w, lin1_b, lin2_w, lin2_b)


def kernel(all_community_embeddings, valid_nodes, index, index1, weight,
           lin1_w, lin1_b, lin2_w, lin2_b):
    x = all_community_embeddings.astype(jnp.float32)
    idx = index.astype(jnp.int32)
    idx1 = index1.astype(jnp.int32)

    pad = jnp.zeros((N_NODES, D_PAD - D_IN), jnp.float32).at[:, 0].set(1.0)
    xpad = jnp.concatenate([x, pad], axis=1)
    zeros = jnp.zeros((ROWS_PER_S, D_PAD), jnp.float32)

    partial = _sc_aggregate(xpad, idx, idx1.reshape(N_EDGES // CHUNK, CHUNK), zeros)
    partial = partial.reshape(NC, N_PAD, D_PAD)
    return _tc_finish(x, partial,
                      weight.astype(jnp.float32),
                      lin1_w.astype(jnp.float32),
                      lin1_b.astype(jnp.float32).reshape(1, D_IN),
                      lin2_w.astype(jnp.float32),
                      lin2_b.astype(jnp.float32).reshape(1, D_IN))


# speedup vs baseline: 1.0686x; 1.0686x over previous
import functools

import jax
import jax.numpy as jnp
from jax import lax
from jax.experimental import pallas as pl
from jax.experimental.pallas import tpu as pltpu
from jax.experimental.pallas import tpu_sc as plsc

N_NODES = 10000
N_PAD = 10240
N_EDGES = 320000
D_IN = 128
D_PAD = 160

NC = 2
NS = 16
NW = NC * NS
E_PER_W = N_EDGES // NW
CHUNK = 80
N_CHUNKS = E_PER_W // CHUNK
N_PAIRS = (N_CHUNKS - 1) // 2
ROWS_PER_S = N_PAD // NS


def _sc_aggregate(xpad, index, index1, zeros):
    mesh = plsc.VectorSubcoreMesh(core_axis_name="c", subcore_axis_name="s")

    @functools.partial(
        pl.kernel,
        mesh=mesh,
        out_type=jax.ShapeDtypeStruct((NC * N_PAD, D_PAD), jnp.bfloat16),
        scratch_types=[
            pltpu.VMEM_SHARED((N_PAD, D_PAD), jnp.bfloat16),
            pltpu.VMEM((2, CHUNK), jnp.int32),
            pltpu.VMEM((N_CHUNKS, CHUNK), jnp.int32),
            pltpu.VMEM((2, CHUNK, D_PAD), jnp.bfloat16),
            pltpu.SemaphoreType.DMA,
            pltpu.SemaphoreType.DMA,
            pltpu.SemaphoreType.DMA,
            pltpu.SemaphoreType.DMA,
        ],
        compiler_params=pltpu.CompilerParams(use_tc_tiling_on_sc=False),
    )
    def k(xpad_h, idx_h, idx1_h, zero_h, out_h,
          acc, idxg, idxs, rows, gs0, gs1, ls0, ls1):
        c = lax.axis_index("c")
        s = lax.axis_index("s")
        wid = c * NS + s

        pltpu.sync_copy(zero_h, acc.at[pl.ds(s * ROWS_PER_S, ROWS_PER_S)])

        ebase = wid * E_PER_W
        pltpu.sync_copy(idx1_h.at[pl.ds(wid * N_CHUNKS, N_CHUNKS)], idxs)
        plsc.subcore_barrier()

        lsem = [ls0, ls1]
        gsem = [gs0, gs1]

        def load_g_start(ch, b):
            pltpu.async_copy(idx_h.at[pl.ds(ebase + ch * CHUNK, CHUNK)],
                             idxg.at[b], lsem[b])

        def load_g_wait(ch, b):
            pltpu.make_async_copy(idx_h.at[pl.ds(ebase + ch * CHUNK, CHUNK)],
                                  idxg.at[b], lsem[b]).wait()

        def gather_start(b):
            pltpu.async_copy(xpad_h.at[idxg.at[b]], rows.at[b], gsem[b])

        def gather_wait(b):
            pltpu.make_async_copy(xpad_h.at[idxg.at[b]], rows.at[b], gsem[b]).wait()

        def scatter(ch, b):
            pltpu.sync_copy(rows.at[b], acc.at[idxs.at[ch]], add=True)

        load_g_start(0, 0)
        load_g_wait(0, 0)
        gather_start(0)
        load_g_start(1, 1)

        def pair_body(i, carry):
            c0 = 2 * i
            c1 = c0 + 1
            gather_wait(0)
            load_g_wait(c1, 1)
            gather_start(1)

            @pl.when(c0 + 2 < N_CHUNKS)
            def _():
                load_g_start(c0 + 2, 0)

            scatter(c0, 0)
            gather_wait(1)

            @pl.when(c0 + 2 < N_CHUNKS)
            def _():
                load_g_wait(c0 + 2, 0)
                gather_start(0)

            @pl.when(c1 + 2 < N_CHUNKS)
            def _():
                load_g_start(c1 + 2, 1)

            scatter(c1, 1)
            return carry

        lax.fori_loop(0, N_PAIRS, pair_body, 0)

        gather_wait(0)
        scatter(N_CHUNKS - 1, 0)
        plsc.subcore_barrier()

        obase = c * N_PAD + s * ROWS_PER_S
        pltpu.sync_copy(acc.at[pl.ds(s * ROWS_PER_S, ROWS_PER_S)],
                        out_h.at[pl.ds(obase, ROWS_PER_S)])

    return k(xpad, index, index1, zeros)


_TC_R = 1000


def _tc_body(x_ref, p0_ref, p1_ref, w_ref, w1_ref, b1_ref, w2_ref, b2_ref, o_ref):
    x = x_ref[...]
    p = p0_ref[0].astype(jnp.float32) + p1_ref[0].astype(jnp.float32)
    aggr_x = p[:, :D_IN]
    deg = p[:, D_IN:D_IN + 1]
    lin1 = jnp.dot(x, w1_ref[...], preferred_element_type=jnp.float32) + b1_ref[...]
    lin2 = jnp.dot(x, w2_ref[...], preferred_element_type=jnp.float32) + b2_ref[...]
    aggr = jnp.dot(aggr_x, w_ref[...], preferred_element_type=jnp.float32)
    o_ref[...] = deg * lin1 + aggr + lin2


def _tc_finish(x, partial, weight, lin1_w, lin1_b, lin2_w, lin2_b):
    grid = N_NODES // _TC_R
    return pl.pallas_call(
        _tc_body,
        grid=(grid,),
        in_specs=[
            pl.BlockSpec((_TC_R, D_IN), lambda i: (i, 0)),
            pl.BlockSpec((1, _TC_R, D_PAD), lambda i: (0, i, 0)),
            pl.BlockSpec((1, _TC_R, D_PAD), lambda i: (1, i, 0)),
            pl.BlockSpec((D_IN, D_IN), lambda i: (0, 0)),
            pl.BlockSpec((D_IN, D_IN), lambda i: (0, 0)),
            pl.BlockSpec((1, D_IN), lambda i: (0, 0)),
            pl.BlockSpec((D_IN, D_IN), lambda i: (0, 0)),
            pl.BlockSpec((1, D_IN), lambda i: (0, 0)),
        ],
        out_specs=pl.BlockSpec((_TC_R, D_IN), lambda i: (i, 0)),
        out_shape=jax.ShapeDtypeStruct((N_NODES, D_IN), jnp.float32),
    )(x, partial, partial, weight, lin1_w, lin1_b, lin2_w, lin2_b)


def kernel(all_community_embeddings, valid_nodes, index, index1, weight,
           lin1_w, lin1_b, lin2_w, lin2_b):
    x = all_community_embeddings.astype(jnp.float32)
    idx = index.astype(jnp.int32)
    idx1 = index1.astype(jnp.int32)

    pad = jnp.zeros((N_NODES, D_PAD - D_IN), jnp.bfloat16).at[:, 0].set(1.0)
    xpad = jnp.concatenate([x.astype(jnp.bfloat16), pad], axis=1)
    zeros = jnp.zeros((ROWS_PER_S, D_PAD), jnp.bfloat16)

    partial = _sc_aggregate(xpad, idx, idx1.reshape(N_EDGES // CHUNK, CHUNK), zeros)
    partial = partial.reshape(NC, N_PAD, D_PAD)
    return _tc_finish(x, partial,
                      weight.astype(jnp.float32),
                      lin1_w.astype(jnp.float32),
                      lin1_b.astype(jnp.float32).reshape(1, D_IN),
                      lin2_w.astype(jnp.float32),
                      lin2_b.astype(jnp.float32).reshape(1, D_IN))


# speedup vs baseline: 1.2567x; 1.1760x over previous
import functools

import jax
import jax.numpy as jnp
from jax import lax
from jax.experimental import pallas as pl
from jax.experimental.pallas import tpu as pltpu
from jax.experimental.pallas import tpu_sc as plsc

N_NODES = 10000
N_PAD = 10240
N_EDGES = 320000
D_IN = 128
D_PAD = 160

NC = 2
NS = 16
NW = NC * NS
E_PER_W = N_EDGES // NW
CHUNK = 80
N_CHUNKS = E_PER_W // CHUNK
NBUF = 5
N_GROUPS = N_CHUNKS // NBUF
ROWS_PER_S = N_PAD // NS


def _sc_aggregate(xpad, index, index1, zeros):
    mesh = plsc.VectorSubcoreMesh(core_axis_name="c", subcore_axis_name="s")

    @functools.partial(
        pl.kernel,
        mesh=mesh,
        out_type=jax.ShapeDtypeStruct((NC * N_PAD, D_PAD), jnp.bfloat16),
        scratch_types=[
            pltpu.VMEM_SHARED((N_PAD, D_PAD), jnp.bfloat16),
            pltpu.VMEM((NBUF, CHUNK), jnp.int32),
            pltpu.VMEM((N_CHUNKS, CHUNK), jnp.int32),
            pltpu.VMEM((NBUF, CHUNK, D_PAD), jnp.bfloat16),
            pltpu.SemaphoreType.DMA,
            pltpu.SemaphoreType.DMA,
            pltpu.SemaphoreType.DMA,
        ],
        compiler_params=pltpu.CompilerParams(use_tc_tiling_on_sc=False),
    )
    def k(xpad_h, idx_h, idx1_h, zero_h, out_h,
          acc, idxg, idxs, rows, lsem, gsem, ssem):
        c = lax.axis_index("c")
        s = lax.axis_index("s")
        wid = c * NS + s

        pltpu.sync_copy(zero_h, acc.at[pl.ds(s * ROWS_PER_S, ROWS_PER_S)])

        ebase = wid * E_PER_W
        pltpu.sync_copy(idx1_h.at[pl.ds(wid * N_CHUNKS, N_CHUNKS)], idxs)
        plsc.subcore_barrier()

        def load_start(ch, b):
            pltpu.async_copy(idx_h.at[pl.ds(ebase + ch * CHUNK, CHUNK)],
                             idxg.at[b], lsem)

        def load_wait(ch, b):
            pltpu.make_async_copy(idx_h.at[pl.ds(ebase + ch * CHUNK, CHUNK)],
                                  idxg.at[b], lsem).wait()

        def gather_start(b):
            pltpu.async_copy(xpad_h.at[idxg.at[b]], rows.at[b], gsem)

        def gather_wait(b):
            pltpu.make_async_copy(xpad_h.at[idxg.at[b]], rows.at[b], gsem).wait()

        def scatter_start(ch, b):
            pltpu.async_copy(rows.at[b], acc.at[idxs.at[ch]], ssem, add=True)

        def scatter_wait(ch, b):
            pltpu.make_async_copy(rows.at[b], acc.at[idxs.at[ch]], ssem).wait()

        for b0 in range(NBUF):
            load_start(b0, b0)
        load_wait(0, 0)
        gather_start(0)

        def step(c, i, u):
            b = u
            bn = (u + 1) % NBUF

            @pl.when(c >= 4)
            def _():
                scatter_wait(c - 4, bn)

            @pl.when(c + 1 < N_CHUNKS)
            def _():
                load_wait(c + 1, bn)
                gather_start(bn)

            gather_wait(b)
            scatter_start(c, b)

            @pl.when(c + NBUF < N_CHUNKS)
            def _():
                load_start(c + NBUF, b)

        def group_body(i, carry):
            for u in range(NBUF):
                step(i * NBUF + u, i, u)
            return carry

        lax.fori_loop(0, N_GROUPS, group_body, 0)

        for ch in range(N_CHUNKS - 4, N_CHUNKS):
            scatter_wait(ch, ch % NBUF)
        plsc.subcore_barrier()

        obase = c * N_PAD + s * ROWS_PER_S
        pltpu.sync_copy(acc.at[pl.ds(s * ROWS_PER_S, ROWS_PER_S)],
                        out_h.at[pl.ds(obase, ROWS_PER_S)])

    return k(xpad, index, index1, zeros)


_TC_R = 1000


def _tc_body(x_ref, p0_ref, p1_ref, w_ref, w1_ref, b1_ref, w2_ref, b2_ref, o_ref):
    x = x_ref[...]
    p = p0_ref[0].astype(jnp.float32) + p1_ref[0].astype(jnp.float32)
    aggr_x = p[:, :D_IN]
    deg = p[:, D_IN:D_IN + 1]
    lin1 = jnp.dot(x, w1_ref[...], preferred_element_type=jnp.float32) + b1_ref[...]
    lin2 = jnp.dot(x, w2_ref[...], preferred_element_type=jnp.float32) + b2_ref[...]
    aggr = jnp.dot(aggr_x, w_ref[...], preferred_element_type=jnp.float32)
    o_ref[...] = deg * lin1 + aggr + lin2


def _tc_finish(x, partial, weight, lin1_w, lin1_b, lin2_w, lin2_b):
    grid = N_NODES // _TC_R
    return pl.pallas_call(
        _tc_body,
        grid=(grid,),
        in_specs=[
            pl.BlockSpec((_TC_R, D_IN), lambda i: (i, 0)),
            pl.BlockSpec((1, _TC_R, D_PAD), lambda i: (0, i, 0)),
            pl.BlockSpec((1, _TC_R, D_PAD), lambda i: (1, i, 0)),
            pl.BlockSpec((D_IN, D_IN), lambda i: (0, 0)),
            pl.BlockSpec((D_IN, D_IN), lambda i: (0, 0)),
            pl.BlockSpec((1, D_IN), lambda i: (0, 0)),
            pl.BlockSpec((D_IN, D_IN), lambda i: (0, 0)),
            pl.BlockSpec((1, D_IN), lambda i: (0, 0)),
        ],
        out_specs=pl.BlockSpec((_TC_R, D_IN), lambda i: (i, 0)),
        out_shape=jax.ShapeDtypeStruct((N_NODES, D_IN), jnp.float32),
    )(x, partial, partial, weight, lin1_w, lin1_b, lin2_w, lin2_b)


def kernel(all_community_embeddings, valid_nodes, index, index1, weight,
           lin1_w, lin1_b, lin2_w, lin2_b):
    x = all_community_embeddings.astype(jnp.float32)
    idx = index.astype(jnp.int32)
    idx1 = index1.astype(jnp.int32)

    pad = jnp.zeros((N_NODES, D_PAD - D_IN), jnp.bfloat16).at[:, 0].set(1.0)
    xpad = jnp.concatenate([x.astype(jnp.bfloat16), pad], axis=1)
    zeros = jnp.zeros((ROWS_PER_S, D_PAD), jnp.bfloat16)

    partial = _sc_aggregate(xpad, idx, idx1.reshape(N_EDGES // CHUNK, CHUNK), zeros)
    partial = partial.reshape(NC, N_PAD, D_PAD)
    return _tc_finish(x, partial,
                      weight.astype(jnp.float32),
                      lin1_w.astype(jnp.float32),
                      lin1_b.astype(jnp.float32).reshape(1, D_IN),
                      lin2_w.astype(jnp.float32),
                      lin2_b.astype(jnp.float32).reshape(1, D_IN))


# speedup vs baseline: 1.4739x; 1.1728x over previous
import functools

import jax
import jax.numpy as jnp
from jax import lax
from jax.experimental import pallas as pl
from jax.experimental.pallas import tpu as pltpu
from jax.experimental.pallas import tpu_sc as plsc

N_NODES = 10000
N_PAD = 10240
N_EDGES = 320000
D_IN = 128
D_DEG = 16

NC = 2
NS = 16
NW = NC * NS
E_PER_W = N_EDGES // NW
CHUNK = 80
N_CHUNKS = E_PER_W // CHUNK
NBUF = 3
N_GROUPS = 41
ROWS_PER_S = N_PAD // NS
DROWS_PER_S = N_PAD // NS


def _sc_aggregate(x, index, index1, zeros, dzeros):
    mesh = plsc.VectorSubcoreMesh(core_axis_name="c", subcore_axis_name="s")

    @functools.partial(
        pl.kernel,
        mesh=mesh,
        out_type=(
            jax.ShapeDtypeStruct((NC * N_PAD, D_IN), jnp.float32),
            jax.ShapeDtypeStruct((NC * N_PAD, D_DEG), jnp.float32),
        ),
        scratch_types=[
            pltpu.VMEM_SHARED((N_PAD, D_IN), jnp.float32),
            pltpu.VMEM_SHARED((N_PAD, D_DEG), jnp.float32),
            pltpu.VMEM((NBUF, CHUNK), jnp.int32),
            pltpu.VMEM((NBUF, CHUNK), jnp.int32),
            pltpu.VMEM((NBUF, CHUNK, D_IN), jnp.float32),
            pltpu.VMEM((CHUNK, D_DEG), jnp.float32),
            pltpu.SemaphoreType.DMA,
            pltpu.SemaphoreType.DMA,
            pltpu.SemaphoreType.DMA,
            pltpu.SemaphoreType.DMA,
            pltpu.SemaphoreType.DMA,
        ],
        compiler_params=pltpu.CompilerParams(use_tc_tiling_on_sc=False),
    )
    def k(x_h, idx_h, idx1_h, zero_h, dzero_h, feat_h, deg_h,
          acc, dacc, idxg, idxs, rows, ones, lgsem, lssem, gsem, ssem, dsem):
        c = lax.axis_index("c")
        s = lax.axis_index("s")
        wid = c * NS + s

        onev = jnp.ones((16,), jnp.float32)
        for r in range(CHUNK):
            ones[r] = onev

        pltpu.sync_copy(zero_h, acc.at[pl.ds(s * ROWS_PER_S, ROWS_PER_S)])
        pltpu.sync_copy(dzero_h, dacc.at[pl.ds(s * DROWS_PER_S, DROWS_PER_S)])
        plsc.subcore_barrier()

        ebase = wid * E_PER_W

        def lg_start(ch, b):
            pltpu.async_copy(idx_h.at[pl.ds(ebase + ch * CHUNK, CHUNK)],
                             idxg.at[b], lgsem)

        def lg_wait(ch, b):
            pltpu.make_async_copy(idx_h.at[pl.ds(ebase + ch * CHUNK, CHUNK)],
                                  idxg.at[b], lgsem).wait()

        def ls_start(ch, b):
            pltpu.async_copy(idx1_h.at[pl.ds(ebase + ch * CHUNK, CHUNK)],
                             idxs.at[b], lssem)

        def ls_wait(ch, b):
            pltpu.make_async_copy(idx1_h.at[pl.ds(ebase + ch * CHUNK, CHUNK)],
                                  idxs.at[b], lssem).wait()

        def g_start(b):
            pltpu.async_copy(x_h.at[idxg.at[b]], rows.at[b], gsem)

        def g_wait(b):
            pltpu.make_async_copy(x_h.at[idxg.at[b]], rows.at[b], gsem).wait()

        def s_start(b):
            pltpu.async_copy(rows.at[b], acc.at[idxs.at[b]], ssem, add=True)
            pltpu.async_copy(ones, dacc.at[idxs.at[b]], dsem, add=True)

        def s_wait(b):
            pltpu.make_async_copy(rows.at[b], acc.at[idxs.at[b]], ssem).wait()
            pltpu.make_async_copy(ones, dacc.at[idxs.at[b]], dsem).wait()

        for b0 in range(NBUF):
            lg_start(b0, b0)
        ls_start(0, 0)
        lg_wait(0, 0)
        g_start(0)

        def body(c_, u):
            b = u
            bn = (u + 1) % NBUF
            if isinstance(c_, int):
                if c_ >= 2:
                    s_wait(bn)
                if c_ + 1 < N_CHUNKS:
                    ls_start(c_ + 1, bn)
                    lg_wait(c_ + 1, bn)
                    g_start(bn)
                g_wait(b)
                if c_ + NBUF < N_CHUNKS:
                    lg_start(c_ + NBUF, b)
                ls_wait(c_, b)
                s_start(b)
            else:
                @pl.when(c_ >= 2)
                def _():
                    s_wait(bn)

                @pl.when(c_ + 1 < N_CHUNKS)
                def _():
                    ls_start(c_ + 1, bn)
                    lg_wait(c_ + 1, bn)
                    g_start(bn)

                g_wait(b)

                @pl.when(c_ + NBUF < N_CHUNKS)
                def _():
                    lg_start(c_ + NBUF, b)

                ls_wait(c_, b)
                s_start(b)

        def group_body(i, carry):
            for u in range(NBUF):
                body(i * NBUF + u, u)
            return carry

        lax.fori_loop(0, N_GROUPS, group_body, 0)
        body(123, 0)
        body(124, 1)
        s_wait(0)
        s_wait(1)
        plsc.subcore_barrier()

        fbase = c * N_PAD + s * ROWS_PER_S
        pltpu.sync_copy(acc.at[pl.ds(s * ROWS_PER_S, ROWS_PER_S)],
                        feat_h.at[pl.ds(fbase, ROWS_PER_S)])
        pltpu.sync_copy(dacc.at[pl.ds(s * DROWS_PER_S, DROWS_PER_S)],
                        deg_h.at[pl.ds(fbase, DROWS_PER_S)])

    return k(x, index, index1, zeros, dzeros)


_TC_R = 640


def _tc_body(x_ref, p0_ref, p1_ref, d0_ref, d1_ref,
             w_ref, w1_ref, b1_ref, w2_ref, b2_ref, o_ref):
    x = x_ref[...]
    aggr_x = p0_ref[...] + p1_ref[...]
    d = d0_ref[...] + d1_ref[...]
    deg = d[:, 0:1]
    lin1 = jnp.dot(x, w1_ref[...], preferred_element_type=jnp.float32) + b1_ref[...]
    lin2 = jnp.dot(x, w2_ref[...], preferred_element_type=jnp.float32) + b2_ref[...]
    aggr = jnp.dot(aggr_x, w_ref[...], preferred_element_type=jnp.float32)
    o_ref[...] = deg * lin1 + aggr + lin2


def _tc_finish(x, feat, deg, weight, lin1_w, lin1_b, lin2_w, lin2_b):
    grid = N_NODES // _TC_R + (1 if N_NODES % _TC_R else 0)
    return pl.pallas_call(
        _tc_body,
        grid=(grid,),
        in_specs=[
            pl.BlockSpec((_TC_R, D_IN), lambda i: (i, 0)),
            pl.BlockSpec((_TC_R, D_IN), lambda i: (i, 0)),
            pl.BlockSpec((_TC_R, D_IN), lambda i: (N_PAD // _TC_R + i, 0)),
            pl.BlockSpec((_TC_R, D_DEG), lambda i: (i, 0)),
            pl.BlockSpec((_TC_R, D_DEG), lambda i: (N_PAD // _TC_R + i, 0)),
            pl.BlockSpec((D_IN, D_IN), lambda i: (0, 0)),
            pl.BlockSpec((D_IN, D_IN), lambda i: (0, 0)),
            pl.BlockSpec((1, D_IN), lambda i: (0, 0)),
            pl.BlockSpec((D_IN, D_IN), lambda i: (0, 0)),
            pl.BlockSpec((1, D_IN), lambda i: (0, 0)),
        ],
        out_specs=pl.BlockSpec((_TC_R, D_IN), lambda i: (i, 0)),
        out_shape=jax.ShapeDtypeStruct((N_NODES, D_IN), jnp.float32),
    )(x, feat, feat, deg, deg, weight, lin1_w, lin1_b, lin2_w, lin2_b)


def kernel(all_community_embeddings, valid_nodes, index, index1, weight,
           lin1_w, lin1_b, lin2_w, lin2_b):
    x = all_community_embeddings.astype(jnp.float32)
    idx = index.astype(jnp.int32)
    idx1 = index1.astype(jnp.int32)
    zeros = jnp.zeros((ROWS_PER_S, D_IN), jnp.float32)
    dzeros = jnp.zeros((DROWS_PER_S, D_DEG), jnp.float32)

    feat, deg = _sc_aggregate(x, idx, idx1, zeros, dzeros)
    return _tc_finish(x, feat, deg,
                      weight.astype(jnp.float32),
                      lin1_w.astype(jnp.float32),
                      lin1_b.astype(jnp.float32).reshape(1, D_IN),
                      lin2_w.astype(jnp.float32),
                      lin2_b.astype(jnp.float32).reshape(1, D_IN))


# speedup vs baseline: 1.5391x; 1.0442x over previous
import functools

import jax
import jax.numpy as jnp
from jax import lax
from jax.experimental import pallas as pl
from jax.experimental.pallas import tpu as pltpu
from jax.experimental.pallas import tpu_sc as plsc

N_NODES = 10000
N_PAD = 10240
N_EDGES = 320000
D_IN = 128
D_DEG = 16

NC = 2
NS = 16
NW = NC * NS
E_PER_W = N_EDGES // NW
CHUNK = 80
N_CHUNKS = E_PER_W // CHUNK
NBUF = 3
N_GROUPS = 41
ROWS_PER_S = N_PAD // NS
DROWS_PER_S = N_PAD // NS


def _sc_aggregate(x, index, index1, zeros, dzeros):
    mesh = plsc.VectorSubcoreMesh(core_axis_name="c", subcore_axis_name="s")

    @functools.partial(
        pl.kernel,
        mesh=mesh,
        out_type=(
            jax.ShapeDtypeStruct((NC * N_PAD, D_IN), jnp.float32),
            jax.ShapeDtypeStruct((NC * N_PAD, D_DEG), jnp.float32),
        ),
        scratch_types=[
            pltpu.VMEM_SHARED((N_PAD, D_IN), jnp.float32),
            pltpu.VMEM_SHARED((N_PAD, D_DEG), jnp.float32),
            pltpu.VMEM((NBUF, CHUNK), jnp.int32),
            pltpu.VMEM((NBUF, CHUNK), jnp.int32),
            pltpu.VMEM((NBUF, CHUNK, D_IN), jnp.float32),
            pltpu.VMEM((CHUNK, D_DEG), jnp.float32),
            pltpu.SemaphoreType.DMA,
            pltpu.SemaphoreType.DMA,
            pltpu.SemaphoreType.DMA,
            pltpu.SemaphoreType.DMA,
            pltpu.SemaphoreType.DMA,
        ],
        compiler_params=pltpu.CompilerParams(use_tc_tiling_on_sc=False),
    )
    def k(x_h, idx_h, idx1_h, zero_h, dzero_h, feat_h, deg_h,
          acc, dacc, idxg, idxs, rows, ones, lgsem, lssem, gsem, ssem, dsem):
        c = lax.axis_index("c")
        s = lax.axis_index("s")
        wid = c * NS + s

        onev = jnp.ones((16,), jnp.float32)
        for r in range(CHUNK):
            ones[r] = onev

        pltpu.sync_copy(zero_h, acc.at[pl.ds(s * ROWS_PER_S, ROWS_PER_S)])
        pltpu.sync_copy(dzero_h, dacc.at[pl.ds(s * DROWS_PER_S, DROWS_PER_S)])
        plsc.subcore_barrier()

        ebase = wid * E_PER_W

        def lg_start(ch, b):
            pltpu.async_copy(idx_h.at[pl.ds(ebase + ch * CHUNK, CHUNK)],
                             idxg.at[b], lgsem)

        def lg_wait(ch, b):
            pltpu.make_async_copy(idx_h.at[pl.ds(ebase + ch * CHUNK, CHUNK)],
                                  idxg.at[b], lgsem).wait()

        def ls_start(ch, b):
            pltpu.async_copy(idx1_h.at[pl.ds(ebase + ch * CHUNK, CHUNK)],
                             idxs.at[b], lssem)

        def ls_wait(ch, b):
            pltpu.make_async_copy(idx1_h.at[pl.ds(ebase + ch * CHUNK, CHUNK)],
                                  idxs.at[b], lssem).wait()

        def g_start(b):
            pltpu.async_copy(x_h.at[idxg.at[b]], rows.at[b], gsem)

        def g_wait(b):
            pltpu.make_async_copy(x_h.at[idxg.at[b]], rows.at[b], gsem).wait()

        def s_start(b):
            pltpu.async_copy(rows.at[b], acc.at[idxs.at[b]], ssem, add=True)
            pltpu.async_copy(ones, dacc.at[idxs.at[b]], dsem, add=True)

        def s_wait(b):
            pltpu.make_async_copy(rows.at[b], acc.at[idxs.at[b]], ssem).wait()
            pltpu.make_async_copy(ones, dacc.at[idxs.at[b]], dsem).wait()

        for b0 in range(NBUF):
            lg_start(b0, b0)
        ls_start(0, 0)
        lg_wait(0, 0)
        g_start(0)

        def body(c_, u):
            b = u
            bn = (u + 1) % NBUF
            if isinstance(c_, int):
                if c_ >= 2:
                    s_wait(bn)
                if c_ + 1 < N_CHUNKS:
                    ls_start(c_ + 1, bn)
                    lg_wait(c_ + 1, bn)
                    g_start(bn)
                g_wait(b)
                if c_ + NBUF < N_CHUNKS:
                    lg_start(c_ + NBUF, b)
                ls_wait(c_, b)
                s_start(b)
            else:
                @pl.when(c_ >= 2)
                def _():
                    s_wait(bn)

                @pl.when(c_ + 1 < N_CHUNKS)
                def _():
                    ls_start(c_ + 1, bn)
                    lg_wait(c_ + 1, bn)
                    g_start(bn)

                g_wait(b)

                @pl.when(c_ + NBUF < N_CHUNKS)
                def _():
                    lg_start(c_ + NBUF, b)

                ls_wait(c_, b)
                s_start(b)

        def group_body(i, carry):
            for u in range(NBUF):
                body(i * NBUF + u, u)
            return carry

        lax.fori_loop(0, N_GROUPS, group_body, 0)
        body(123, 0)
        body(124, 1)
        s_wait(0)
        s_wait(1)
        plsc.subcore_barrier()

        fbase = c * N_PAD + s * ROWS_PER_S
        pltpu.sync_copy(acc.at[pl.ds(s * ROWS_PER_S, ROWS_PER_S)],
                        feat_h.at[pl.ds(fbase, ROWS_PER_S)])
        pltpu.sync_copy(dacc.at[pl.ds(s * DROWS_PER_S, DROWS_PER_S)],
                        deg_h.at[pl.ds(fbase, DROWS_PER_S)])

    return k(x, index, index1, zeros, dzeros)


_TC_R = 640
_DEG_ROWS = _TC_R * D_DEG // D_IN


def _tc_lin_body(x_ref, w_ref, b_ref, o_ref):
    o_ref[...] = (jnp.dot(x_ref[...], w_ref[...],
                          preferred_element_type=jnp.float32) + b_ref[...])


def _tc_lin(x, wlin, blin):
    grid = N_NODES // _TC_R + (1 if N_NODES % _TC_R else 0)
    return pl.pallas_call(
        _tc_lin_body,
        grid=(grid,),
        in_specs=[
            pl.BlockSpec((_TC_R, D_IN), lambda i: (i, 0)),
            pl.BlockSpec((D_IN, 2 * D_IN), lambda i: (0, 0)),
            pl.BlockSpec((1, 2 * D_IN), lambda i: (0, 0)),
        ],
        out_specs=pl.BlockSpec((_TC_R, 2 * D_IN), lambda i: (i, 0)),
        out_shape=jax.ShapeDtypeStruct((N_NODES, 2 * D_IN), jnp.float32),
    )(x, wlin, blin)


def _tc_body(lin_ref, p0_ref, p1_ref, d0_ref, d1_ref, w_ref, o_ref):
    lin = lin_ref[...]
    aggr_x = p0_ref[...] + p1_ref[...]
    dblk = d0_ref[...] + d1_ref[...]
    ri = lax.broadcasted_iota(jnp.int32, (_TC_R, _DEG_ROWS), 0)
    ci = lax.broadcasted_iota(jnp.int32, (_TC_R, _DEG_ROWS), 1)
    sel = (ci == ri // 8).astype(jnp.float32)
    rep = jnp.dot(sel, dblk, preferred_element_type=jnp.float32)
    ni = lax.broadcasted_iota(jnp.int32, (_TC_R, D_IN), 0)
    li = lax.broadcasted_iota(jnp.int32, (_TC_R, D_IN), 1)
    cm = (li == (ni % 8) * D_DEG).astype(jnp.float32)
    deg = jnp.sum(rep * cm, axis=1, keepdims=True)
    aggr = jnp.dot(aggr_x, w_ref[...], preferred_element_type=jnp.float32)
    o_ref[...] = deg * lin[:, :D_IN] + aggr + lin[:, D_IN:]


def _tc_finish(lin, feat, degv, weight):
    grid = N_NODES // _TC_R + (1 if N_NODES % _TC_R else 0)
    return pl.pallas_call(
        _tc_body,
        grid=(grid,),
        in_specs=[
            pl.BlockSpec((_TC_R, 2 * D_IN), lambda i: (i, 0)),
            pl.BlockSpec((_TC_R, D_IN), lambda i: (i, 0)),
            pl.BlockSpec((_TC_R, D_IN), lambda i: (N_PAD // _TC_R + i, 0)),
            pl.BlockSpec((_DEG_ROWS, D_IN), lambda i: (i, 0)),
            pl.BlockSpec((_DEG_ROWS, D_IN), lambda i: (N_PAD // _TC_R + i, 0)),
            pl.BlockSpec((D_IN, D_IN), lambda i: (0, 0)),
        ],
        out_specs=pl.BlockSpec((_TC_R, D_IN), lambda i: (i, 0)),
        out_shape=jax.ShapeDtypeStruct((N_NODES, D_IN), jnp.float32),
    )(lin, feat, feat, degv, degv, weight)


def kernel(all_community_embeddings, valid_nodes, index, index1, weight,
           lin1_w, lin1_b, lin2_w, lin2_b):
    x = all_community_embeddings.astype(jnp.float32)
    idx = index.astype(jnp.int32)
    idx1 = index1.astype(jnp.int32)
    zeros = jnp.zeros((ROWS_PER_S, D_IN), jnp.float32)
    dzeros = jnp.zeros((DROWS_PER_S, D_DEG), jnp.float32)

    wlin = jnp.concatenate([lin1_w.astype(jnp.float32),
                            lin2_w.astype(jnp.float32)], axis=1)
    blin = jnp.concatenate([lin1_b.astype(jnp.float32),
                            lin2_b.astype(jnp.float32)]).reshape(1, 2 * D_IN)
    lin = _tc_lin(x, wlin, blin)

    feat, deg = _sc_aggregate(x, idx, idx1, zeros, dzeros)
    degv = deg.reshape(NC * N_PAD * D_DEG // D_IN, D_IN)
    return _tc_finish(lin, feat, degv, weight.astype(jnp.float32))
